# Initial kernel scaffold; baseline (speedup 1.0000x reference)
import jax
import jax.numpy as jnp
from jax.experimental import pallas as pl


def kernel(scores, gold):
    raise NotImplementedError("write your pallas kernel here")


# speedup vs baseline: 3.4415x; 3.4415x over previous
import functools

import jax
import jax.numpy as jnp
from jax.experimental import pallas as pl
from jax.experimental.pallas import tpu as pltpu

_MARGIN = 0.1
_IGNORE_INDEX = 0


def _loss_kernel(x_ref, g_ref, o_ref, neg_acc, gold_acc, *, bw, v, b, nb):
    j = pl.program_id(0)

    @pl.when(j == 0)
    def _init():
        neg_acc[...] = jnp.full_like(neg_acc, -jnp.inf)
        gold_acc[...] = jnp.zeros_like(gold_acc)

    x = x_ref[...]
    col = j * bw + jax.lax.broadcasted_iota(jnp.int32, x.shape, 1)
    g = g_ref[...]
    is_gold = col == g
    invalid = col >= v
    neg = jnp.max(jnp.where(is_gold | invalid, -jnp.inf, x), axis=1, keepdims=True)
    neg_acc[...] = jnp.maximum(neg_acc[...], neg)
    gold_acc[...] += jnp.sum(jnp.where(is_gold, x, 0.0), axis=1, keepdims=True)

    @pl.when(j == nb - 1)
    def _final():
        loss = jnp.maximum(_MARGIN + neg_acc[...] - gold_acc[...], 0.0)
        loss = loss * (g != _IGNORE_INDEX).astype(loss.dtype)
        o_ref[0, 0] = jnp.sum(loss) / b


@functools.partial(jax.jit, static_argnames=("interpret",))
def kernel(scores, gold, interpret=False):
    b, v = scores.shape
    bw = 2048
    nb = pl.cdiv(v, bw)
    gold2 = gold.astype(jnp.int32).reshape(b, 1)
    out = pl.pallas_call(
        functools.partial(_loss_kernel, bw=bw, v=v, b=b, nb=nb),
        grid=(nb,),
        in_specs=[
            pl.BlockSpec((b, bw), lambda j: (0, j)),
            pl.BlockSpec((b, 1), lambda j: (0, 0)),
        ],
        out_specs=pl.BlockSpec(memory_space=pltpu.SMEM),
        out_shape=jax.ShapeDtypeStruct((1, 1), jnp.float32),
        scratch_shapes=[
            pltpu.VMEM((b, 1), jnp.float32),
            pltpu.VMEM((b, 1), jnp.float32),
        ],
        interpret=interpret,
    )(scores, gold2)
    return out[0, 0]
